# baseline (speedup 1.0000x reference)
import functools

import jax
import jax.numpy as jnp
from jax import lax
from jax.experimental import pallas as pl
from jax.experimental.pallas import tpu as pltpu
from jax.experimental.pallas import tpu_sc as plsc

_B = 4
_S = 8192
_D = 1024
_NW = 32
_ROWS_PER_W = (_B * _S) // _NW
_R = 32
_CHUNKS = _ROWS_PER_W // _R
_VECS = _R * (_D // 16)


def _sc_add(x_hbm, table_hbm, out_hbm, bx, bt):
    c = lax.axis_index("c")
    s = lax.axis_index("s")
    wid = s * 2 + c
    base_row = wid * _ROWS_PER_W

    def chunk_body(ci, carry):
        row0 = base_row + ci * _R
        s0 = lax.rem(row0, _S)
        pltpu.sync_copy(x_hbm.at[pl.ds(row0, _R)], bx)
        pltpu.sync_copy(table_hbm.at[pl.ds(s0, _R)], bt)

        def add_body(i, carry2):
            r = i // (_D // 16)
            col = (i % (_D // 16)) * 16
            plsc.addupdate(bx.at[r, pl.ds(col, 16)], bt[r, pl.ds(col, 16)])
            return carry2

        lax.fori_loop(0, _VECS, add_body, 0)
        pltpu.sync_copy(bx, out_hbm.at[pl.ds(row0, _R)])
        return carry

    lax.fori_loop(0, _CHUNKS, chunk_body, 0)


def kernel(x, pos_table):
    batch, seq_len, d_model = x.shape
    x2 = x.reshape(batch * seq_len, d_model)
    mesh = plsc.VectorSubcoreMesh(core_axis_name="c", subcore_axis_name="s")
    k = functools.partial(
        pl.kernel,
        mesh=mesh,
        out_type=jax.ShapeDtypeStruct((batch * seq_len, d_model), jnp.float32),
        scratch_types=[
            pltpu.VMEM((_R, _D), jnp.float32),
            pltpu.VMEM((_R, _D), jnp.float32),
        ],
    )(_sc_add)
    out2 = k(x2, pos_table)
    return out2.reshape(batch, seq_len, d_model)


# speedup vs baseline: 1.4126x; 1.4126x over previous
import functools

import jax
import jax.numpy as jnp
from jax import lax
from jax.experimental import pallas as pl
from jax.experimental.pallas import tpu as pltpu
from jax.experimental.pallas import tpu_sc as plsc

_B = 4
_S = 8192
_D = 1024
_NW = 32
_ROWS_PER_W = (_B * _S) // _NW
_R = 16
_CHUNKS = _ROWS_PER_W // _R
_PAIRS = _CHUNKS // 2
_VECS4 = _R * (_D // 16) // 4


def _add_chunk(bx, bt):
    def add_body(i, carry):
        r = i // 16
        col = (i % 16) * 64
        plsc.addupdate(bx.at[r, pl.ds(col, 16)], bt[r, pl.ds(col, 16)])
        plsc.addupdate(bx.at[r, pl.ds(col + 16, 16)], bt[r, pl.ds(col + 16, 16)])
        plsc.addupdate(bx.at[r, pl.ds(col + 32, 16)], bt[r, pl.ds(col + 32, 16)])
        plsc.addupdate(bx.at[r, pl.ds(col + 48, 16)], bt[r, pl.ds(col + 48, 16)])
        return carry

    lax.fori_loop(0, _VECS4, add_body, 0)


def _sc_add(x_hbm, table_hbm, out_hbm, bxa, bta, bxb, btb,
            sem_in_a, sem_in_b, sem_out_a, sem_out_b):
    c = lax.axis_index("c")
    s = lax.axis_index("s")
    wid = s * 2 + c
    base_row = wid * _ROWS_PER_W

    def start_in(row0, bx, bt, sem):
        s0 = lax.rem(row0, _S)
        pltpu.make_async_copy(x_hbm.at[pl.ds(row0, _R)], bx, sem).start()
        pltpu.make_async_copy(table_hbm.at[pl.ds(s0, _R)], bt, sem).start()

    def wait_in(row0, bx, bt, sem):
        s0 = lax.rem(row0, _S)
        pltpu.make_async_copy(x_hbm.at[pl.ds(row0, _R)], bx, sem).wait()
        pltpu.make_async_copy(table_hbm.at[pl.ds(s0, _R)], bt, sem).wait()

    start_in(base_row, bxa, bta, sem_in_a)

    def pair_body(g, carry):
        row_a = base_row + (2 * g) * _R
        row_b = row_a + _R

        @pl.when(g > 0)
        def _():
            pltpu.make_async_copy(
                bxb, out_hbm.at[pl.ds(row_b - 2 * _R, _R)], sem_out_b).wait()

        start_in(row_b, bxb, btb, sem_in_b)

        wait_in(row_a, bxa, bta, sem_in_a)
        _add_chunk(bxa, bta)
        pltpu.make_async_copy(bxa, out_hbm.at[pl.ds(row_a, _R)], sem_out_a).start()

        wait_in(row_b, bxb, btb, sem_in_b)
        _add_chunk(bxb, btb)
        pltpu.make_async_copy(bxb, out_hbm.at[pl.ds(row_b, _R)], sem_out_b).start()

        pltpu.make_async_copy(bxa, out_hbm.at[pl.ds(row_a, _R)], sem_out_a).wait()

        @pl.when(g < _PAIRS - 1)
        def _():
            start_in(row_a + 2 * _R, bxa, bta, sem_in_a)

        return carry

    lax.fori_loop(0, _PAIRS, pair_body, 0)

    last_row_b = base_row + _ROWS_PER_W - _R
    pltpu.make_async_copy(bxb, out_hbm.at[pl.ds(last_row_b, _R)], sem_out_b).wait()


def kernel(x, pos_table):
    batch, seq_len, d_model = x.shape
    x2 = x.reshape(batch * seq_len, d_model)
    mesh = plsc.VectorSubcoreMesh(core_axis_name="c", subcore_axis_name="s")
    k = functools.partial(
        pl.kernel,
        mesh=mesh,
        out_type=jax.ShapeDtypeStruct((batch * seq_len, d_model), jnp.float32),
        scratch_types=[
            pltpu.VMEM((_R, _D), jnp.float32),
            pltpu.VMEM((_R, _D), jnp.float32),
            pltpu.VMEM((_R, _D), jnp.float32),
            pltpu.VMEM((_R, _D), jnp.float32),
            pltpu.SemaphoreType.DMA,
            pltpu.SemaphoreType.DMA,
            pltpu.SemaphoreType.DMA,
            pltpu.SemaphoreType.DMA,
        ],
    )(_sc_add)
    out2 = k(x2, pos_table)
    return out2.reshape(batch, seq_len, d_model)


# speedup vs baseline: 4.9175x; 3.4813x over previous
import jax
import jax.numpy as jnp
from jax.experimental import pallas as pl
from jax.experimental.pallas import tpu as pltpu

_BLOCK_S = 2048


def _add_kernel(x_ref, pos_ref, out_ref):
    out_ref[0] = x_ref[0] + pos_ref[...]


def kernel(x, pos_table):
    batch, seq_len, d_model = x.shape
    bs = _BLOCK_S
    grid = (seq_len // bs, batch)
    return pl.pallas_call(
        _add_kernel,
        grid=grid,
        in_specs=[
            pl.BlockSpec((1, bs, d_model), lambda i, j: (j, i, 0)),
            pl.BlockSpec((bs, d_model), lambda i, j: (i, 0)),
        ],
        out_specs=pl.BlockSpec((1, bs, d_model), lambda i, j: (j, i, 0)),
        out_shape=jax.ShapeDtypeStruct(x.shape, x.dtype),
        compiler_params=pltpu.CompilerParams(
            dimension_semantics=("parallel", "parallel"),
        ),
    )(x, pos_table)
